# Initial kernel scaffold; baseline (speedup 1.0000x reference)
import jax
import jax.numpy as jnp
from jax.experimental import pallas as pl


def kernel(X, edge_index, edge_vals, W_pass, b_pass, W_self, b_self):
    raise NotImplementedError("write your pallas kernel here")


# speedup vs baseline: 2.6626x; 2.6626x over previous
import functools

import jax
import jax.numpy as jnp
from jax import lax
from jax.experimental import pallas as pl
from jax.experimental.pallas import tpu as pltpu
from jax.experimental.pallas import tpu_sc as plsc

_NW = 32
_C = 512
_SUB = 128
_BR = 8000


def _y_body(x_ref, wya_ref, wyb_ref, ya_ref, yb_ref):
    x = x_ref[...]
    dn = (((1,), (1,)), ((), ()))
    ya_ref[...] = lax.dot_general(x, wya_ref[...], dn,
                                  preferred_element_type=jnp.float32)
    yb_ref[...] = lax.dot_general(x, wyb_ref[...], dn,
                                  preferred_element_type=jnp.float32)


def _z_body(ev_ref, w_ref, b_ref, z_ref):
    dn = (((1,), (1,)), ((), ()))
    z_ref[...] = lax.dot_general(ev_ref[...], w_ref[...], dn,
                                 preferred_element_type=jnp.float32) + b_ref[...]


def _sc_combine(ya, yb, src, dst, z):
    n_edges = src.shape[0]
    d_out = ya.shape[1]
    nchunk = n_edges // _C
    mesh = plsc.VectorSubcoreMesh(core_axis_name="c", subcore_axis_name="s")

    @functools.partial(
        pl.kernel,
        mesh=mesh,
        compiler_params=pltpu.CompilerParams(use_tc_tiling_on_sc=False),
        out_type=jax.ShapeDtypeStruct((n_edges, d_out), jnp.float32),
        scratch_types=[
            pltpu.VMEM((_C,), jnp.int32),
            pltpu.VMEM((_C,), jnp.int32),
            pltpu.VMEM((_C, d_out), jnp.float32),
            pltpu.VMEM((_C, d_out), jnp.float32),
            pltpu.VMEM((_C, d_out), jnp.float32),
            pltpu.SemaphoreType.DMA,
            pltpu.SemaphoreType.DMA,
        ],
    )
    def run(ya_hbm, yb_hbm, src_hbm, dst_hbm, z_hbm, out_hbm,
            ia, ib, ra, rb, rz, sema, semb):
        wid = lax.axis_index("s") * 2 + lax.axis_index("c")

        @pl.loop(wid, nchunk, step=_NW)
        def _chunk(t):
            base = pl.multiple_of(t * _C, 8)
            pltpu.sync_copy(src_hbm.at[pl.ds(base, _C)], ia)
            pltpu.sync_copy(dst_hbm.at[pl.ds(base, _C)], ib)
            copies = []
            for j in range(_C // _SUB):
                o = j * _SUB
                copies.append(pltpu.async_copy(
                    ya_hbm.at[ia.at[pl.ds(o, _SUB)]], ra.at[pl.ds(o, _SUB)],
                    sema))
                copies.append(pltpu.async_copy(
                    yb_hbm.at[ib.at[pl.ds(o, _SUB)]], rb.at[pl.ds(o, _SUB)],
                    semb))
            pltpu.sync_copy(z_hbm.at[pl.ds(base, _C)], rz)
            for c in copies:
                c.wait()

            @pl.loop(0, _C)
            def _edge(i):
                ra[i] = jnp.maximum(ra[i] + rb[i] + rz[i], 0.0)

            pltpu.sync_copy(ra, out_hbm.at[pl.ds(base, _C)])

    return run(ya, yb, src, dst, z)


def kernel(X, edge_index, edge_vals, W_pass, b_pass, W_self, b_self):
    n_nodes, d_n = X.shape
    n_edges = edge_vals.shape[0]
    d_out = W_pass.shape[0]

    src = edge_index[0].astype(jnp.int32)
    dst = edge_index[1].astype(jnp.int32)
    wa = W_pass[:, :d_n]
    wb = W_pass[:, d_n:]
    wya = (wa + wb) * 0.5
    wyb = (wb - wa) * 0.5
    bias = (b_pass + b_self).reshape(1, d_out)

    ya, yb = pl.pallas_call(
        _y_body,
        out_shape=[jax.ShapeDtypeStruct((n_nodes, d_out), jnp.float32)] * 2,
    )(X, wya, wyb)

    z = pl.pallas_call(
        _z_body,
        grid=(n_edges // _BR,),
        in_specs=[
            pl.BlockSpec((_BR, edge_vals.shape[1]), lambda i: (i, 0)),
            pl.BlockSpec(W_self.shape, lambda i: (0, 0)),
            pl.BlockSpec((1, d_out), lambda i: (0, 0)),
        ],
        out_specs=pl.BlockSpec((_BR, d_out), lambda i: (i, 0)),
        out_shape=jax.ShapeDtypeStruct((n_edges, d_out), jnp.float32),
    )(edge_vals, W_self, bias)

    return _sc_combine(ya, yb, src, dst, z)


# speedup vs baseline: 3.4650x; 1.3014x over previous
import functools

import jax
import jax.numpy as jnp
from jax import lax
from jax.experimental import pallas as pl
from jax.experimental.pallas import tpu as pltpu
from jax.experimental.pallas import tpu_sc as plsc

_NW = 32
_C = 512
_SUB = 128
_BR = 8000


def _y_body(x_ref, wya_ref, wyb_ref, ya_ref, yb_ref):
    x = x_ref[...]
    dn = (((1,), (0,)), ((), ()))
    ya_ref[...] = lax.dot_general(x, wya_ref[...], dn,
                                  preferred_element_type=jnp.float32)
    yb_ref[...] = lax.dot_general(x, wyb_ref[...], dn,
                                  preferred_element_type=jnp.float32)


def _z_body(ev_ref, w_ref, b_ref, z_ref):
    dn = (((1,), (0,)), ((), ()))
    z_ref[...] = lax.dot_general(ev_ref[...], w_ref[...], dn,
                                 preferred_element_type=jnp.float32) + b_ref[...]


def _sc_combine(ya, yb, src, dst, z):
    n_edges = src.shape[0]
    d_out = ya.shape[1]
    nchunk = n_edges // _C
    mesh = plsc.VectorSubcoreMesh(core_axis_name="c", subcore_axis_name="s")

    @functools.partial(
        pl.kernel,
        mesh=mesh,
        compiler_params=pltpu.CompilerParams(use_tc_tiling_on_sc=False),
        out_type=jax.ShapeDtypeStruct((n_edges, d_out), jnp.float32),
        scratch_types=[
            pltpu.VMEM((_C,), jnp.int32),
            pltpu.VMEM((_C,), jnp.int32),
            pltpu.VMEM((_C, d_out), jnp.float32),
            pltpu.VMEM((_C, d_out), jnp.float32),
            pltpu.VMEM((_C, d_out), jnp.float32),
            pltpu.SemaphoreType.DMA,
            pltpu.SemaphoreType.DMA,
        ],
    )
    def run(ya_hbm, yb_hbm, src_hbm, dst_hbm, z_hbm, out_hbm,
            ia, ib, ra, rb, rz, sema, semb):
        wid = lax.axis_index("s") * 2 + lax.axis_index("c")

        @pl.loop(wid, nchunk, step=_NW)
        def _chunk(t):
            base = pl.multiple_of(t * _C, 8)
            pltpu.sync_copy(src_hbm.at[pl.ds(base, _C)], ia)
            pltpu.sync_copy(dst_hbm.at[pl.ds(base, _C)], ib)
            copies = []
            for j in range(_C // _SUB):
                o = j * _SUB
                copies.append(pltpu.async_copy(
                    ya_hbm.at[ia.at[pl.ds(o, _SUB)]], ra.at[pl.ds(o, _SUB)],
                    sema))
                copies.append(pltpu.async_copy(
                    yb_hbm.at[ib.at[pl.ds(o, _SUB)]], rb.at[pl.ds(o, _SUB)],
                    semb))
            pltpu.sync_copy(z_hbm.at[pl.ds(base, _C)], rz)
            for c in copies:
                c.wait()

            @pl.loop(0, _C)
            def _edge(i):
                ra[i] = jnp.maximum(ra[i] + rb[i] + rz[i], 0.0)

            pltpu.sync_copy(ra, out_hbm.at[pl.ds(base, _C)])

    return run(ya, yb, src, dst, z)


def kernel(X, edge_index, edge_vals, W_pass, b_pass, W_self, b_self):
    n_nodes, d_n = X.shape
    n_edges = edge_vals.shape[0]
    d_out = W_pass.shape[0]

    d_e = edge_vals.shape[1]

    src = edge_index[0].astype(jnp.int32)
    dst = edge_index[1].astype(jnp.int32)
    wa = W_pass[:, :d_n]
    wb = W_pass[:, d_n:]
    wya = (wa + wb) * 0.5
    wyb = (wb - wa) * 0.5
    bias = (b_pass + b_self).reshape(1, d_out)

    eye8 = jnp.eye(8, dtype=jnp.float32)
    wya_big = jnp.kron(eye8, wya.T)
    wyb_big = jnp.kron(eye8, wyb.T)
    ws_big = jnp.kron(eye8, W_self.T)
    bias_big = jnp.tile(bias, (1, 8))

    xr = X.reshape(n_nodes // 8, 8 * d_n)
    ya2, yb2 = pl.pallas_call(
        _y_body,
        out_shape=[jax.ShapeDtypeStruct((n_nodes // 8, 8 * d_out),
                                        jnp.float32)] * 2,
    )(xr, wya_big, wyb_big)
    ya = ya2.reshape(n_nodes, d_out)
    yb = yb2.reshape(n_nodes, d_out)

    evr = edge_vals.reshape(n_edges // 8, 8 * d_e)
    brows = _BR // 8
    z2 = pl.pallas_call(
        _z_body,
        grid=(n_edges // _BR,),
        in_specs=[
            pl.BlockSpec((brows, 8 * d_e), lambda i: (i, 0)),
            pl.BlockSpec((8 * d_e, 8 * d_out), lambda i: (0, 0)),
            pl.BlockSpec((1, 8 * d_out), lambda i: (0, 0)),
        ],
        out_specs=pl.BlockSpec((brows, 8 * d_out), lambda i: (i, 0)),
        out_shape=jax.ShapeDtypeStruct((n_edges // 8, 8 * d_out),
                                       jnp.float32),
    )(evr, ws_big, bias_big)
    z = z2.reshape(n_edges, d_out)

    return _sc_combine(ya, yb, src, dst, z)


# speedup vs baseline: 3.8791x; 1.1195x over previous
import functools

import jax
import jax.numpy as jnp
from jax import lax
from jax.experimental import pallas as pl
from jax.experimental.pallas import tpu as pltpu
from jax.experimental.pallas import tpu_sc as plsc

_NW = 32
_C = 512
_SUB = 128
_BR = 8000


def _y_body(x_ref, wya_ref, wyb_ref, ya_ref, yb_ref):
    x = x_ref[...]
    dn = (((1,), (0,)), ((), ()))
    ya_ref[...] = lax.dot_general(x, wya_ref[...], dn,
                                  preferred_element_type=jnp.float32)
    yb_ref[...] = lax.dot_general(x, wyb_ref[...], dn,
                                  preferred_element_type=jnp.float32)


def _z_body(ev_ref, w_ref, b_ref, z_ref):
    dn = (((1,), (0,)), ((), ()))
    z_ref[...] = lax.dot_general(ev_ref[...], w_ref[...], dn,
                                 preferred_element_type=jnp.float32) + b_ref[...]


def _sc_combine(ya, yb, src, dst, z2):
    n_edges = src.shape[0]
    d_out = ya.shape[1]
    nchunk = n_edges // _C
    crows = _C // 8
    mesh = plsc.VectorSubcoreMesh(core_axis_name="c", subcore_axis_name="s")

    @functools.partial(
        pl.kernel,
        mesh=mesh,
        compiler_params=pltpu.CompilerParams(use_tc_tiling_on_sc=False),
        out_type=jax.ShapeDtypeStruct((n_edges // 8, 8 * d_out), jnp.float32),
        scratch_types=[
            pltpu.VMEM((_C,), jnp.int32),
            pltpu.VMEM((_C,), jnp.int32),
            pltpu.VMEM((_C, d_out), jnp.float32),
            pltpu.VMEM((_C, d_out), jnp.float32),
            pltpu.VMEM((crows, 8 * d_out), jnp.float32),
            pltpu.SemaphoreType.DMA,
            pltpu.SemaphoreType.DMA,
        ],
    )
    def run(ya_hbm, yb_hbm, src_hbm, dst_hbm, z_hbm, out_hbm,
            ia, ib, ra, rb, rzo, sema, semb):
        wid = lax.axis_index("s") * 2 + lax.axis_index("c")

        @pl.loop(wid, nchunk, step=_NW)
        def _chunk(t):
            base = pl.multiple_of(t * _C, 8)
            brow = pl.multiple_of(t * crows, 8)
            pltpu.sync_copy(src_hbm.at[pl.ds(base, _C)], ia)
            pltpu.sync_copy(dst_hbm.at[pl.ds(base, _C)], ib)
            copies = []
            for j in range(_C // _SUB):
                o = j * _SUB
                copies.append(pltpu.async_copy(
                    ya_hbm.at[ia.at[pl.ds(o, _SUB)]], ra.at[pl.ds(o, _SUB)],
                    sema))
                copies.append(pltpu.async_copy(
                    yb_hbm.at[ib.at[pl.ds(o, _SUB)]], rb.at[pl.ds(o, _SUB)],
                    semb))
            pltpu.sync_copy(z_hbm.at[pl.ds(brow, crows)], rzo)
            for c in copies:
                c.wait()

            @pl.loop(0, crows)
            def _row(r):
                e = r * 8
                for j in range(8):
                    s = ra[e + j] + rb[e + j]
                    c16 = pl.ds(j * d_out, d_out)
                    rzo[r, c16] = jnp.maximum(s + rzo[r, c16], 0.0)

            pltpu.sync_copy(rzo, out_hbm.at[pl.ds(brow, crows)])

    return run(ya, yb, src, dst, z2)


def kernel(X, edge_index, edge_vals, W_pass, b_pass, W_self, b_self):
    n_nodes, d_n = X.shape
    n_edges = edge_vals.shape[0]
    d_out = W_pass.shape[0]

    d_e = edge_vals.shape[1]

    src = edge_index[0].astype(jnp.int32)
    dst = edge_index[1].astype(jnp.int32)
    wa = W_pass[:, :d_n]
    wb = W_pass[:, d_n:]
    wya = (wa + wb) * 0.5
    wyb = (wb - wa) * 0.5
    bias = (b_pass + b_self).reshape(1, d_out)

    eye8 = jnp.eye(8, dtype=jnp.float32)
    wya_big = jnp.kron(eye8, wya.T)
    wyb_big = jnp.kron(eye8, wyb.T)
    ws_big = jnp.kron(eye8, W_self.T)
    bias_big = jnp.tile(bias, (1, 8))

    xr = X.reshape(n_nodes // 8, 8 * d_n)
    ya2, yb2 = pl.pallas_call(
        _y_body,
        out_shape=[jax.ShapeDtypeStruct((n_nodes // 8, 8 * d_out),
                                        jnp.float32)] * 2,
    )(xr, wya_big, wyb_big)
    ya = ya2.reshape(n_nodes, d_out)
    yb = yb2.reshape(n_nodes, d_out)

    evr = edge_vals.reshape(n_edges // 8, 8 * d_e)
    brows = _BR // 8
    z2 = pl.pallas_call(
        _z_body,
        grid=(n_edges // _BR,),
        in_specs=[
            pl.BlockSpec((brows, 8 * d_e), lambda i: (i, 0)),
            pl.BlockSpec((8 * d_e, 8 * d_out), lambda i: (0, 0)),
            pl.BlockSpec((1, 8 * d_out), lambda i: (0, 0)),
        ],
        out_specs=pl.BlockSpec((brows, 8 * d_out), lambda i: (i, 0)),
        out_shape=jax.ShapeDtypeStruct((n_edges // 8, 8 * d_out),
                                       jnp.float32),
    )(evr, ws_big, bias_big)

    out2 = _sc_combine(ya, yb, src, dst, z2)
    return out2.reshape(n_edges, d_out)


# speedup vs baseline: 4.6105x; 1.1885x over previous
import functools

import jax
import jax.numpy as jnp
from jax import lax
from jax.experimental import pallas as pl
from jax.experimental.pallas import tpu as pltpu
from jax.experimental.pallas import tpu_sc as plsc

_NW = 32
_C = 512
_SUB = 128
_BC = 12800


def _y_body(x_ref, wya_ref, wyb_ref, b_ref, ya_ref, yb_ref):
    x = x_ref[...]
    dn = (((1,), (0,)), ((), ()))
    ya_ref[...] = lax.dot_general(x, wya_ref[...], dn,
                                  preferred_element_type=jnp.float32) + b_ref[...]
    yb_ref[...] = lax.dot_general(x, wyb_ref[...], dn,
                                  preferred_element_type=jnp.float32)


def _zt_body(evt_ref, w_ref, z_ref):
    zt = lax.dot_general(w_ref[...], evt_ref[...], (((1,), (0,)), ((), ())),
                         preferred_element_type=jnp.float32)
    bt = zt.shape[1] // 128
    z_ref[...] = zt.reshape(2, 8, bt, 128).transpose(0, 2, 1, 3)


def _sc_combine(ya, yb, src, dst, zb):
    n_edges = src.shape[0]
    d_out = ya.shape[1]
    nchunk = n_edges // _C
    ctiles = _C // 128
    ntiles = n_edges // 128
    mesh = plsc.VectorSubcoreMesh(core_axis_name="c", subcore_axis_name="s")

    @functools.partial(
        pl.kernel,
        mesh=mesh,
        compiler_params=pltpu.CompilerParams(use_tc_tiling_on_sc=False,
                                             needs_layout_passes=False),
        out_type=jax.ShapeDtypeStruct((2, ntiles, 8, 128), jnp.float32),
        scratch_types=[
            pltpu.VMEM((_C,), jnp.int32),
            pltpu.VMEM((_C,), jnp.int32),
            pltpu.VMEM((_C, d_out), jnp.float32),
            pltpu.VMEM((_C, d_out), jnp.float32),
            pltpu.VMEM((ctiles, 16, 128), jnp.float32),
            pltpu.VMEM((ctiles, 16, 128), jnp.float32),
            pltpu.SemaphoreType.DMA,
            pltpu.SemaphoreType.DMA,
        ],
    )
    def run(ya_hbm, yb_hbm, src_hbm, dst_hbm, z_hbm, out_hbm,
            ia, ib, ra, rb, rz, rt, sema, semb):
        wid = lax.axis_index("s") * 2 + lax.axis_index("c")
        row_iota = lax.iota(jnp.int32, 16)

        @pl.loop(wid, nchunk, step=_NW)
        def _chunk(t):
            base = pl.multiple_of(t * _C, 8)
            tb = pl.multiple_of(t * ctiles, 4)
            pltpu.sync_copy(src_hbm.at[pl.ds(base, _C)], ia)
            pltpu.sync_copy(dst_hbm.at[pl.ds(base, _C)], ib)
            copies = []
            for j in range(_C // _SUB):
                o = j * _SUB
                copies.append(pltpu.async_copy(
                    ya_hbm.at[ia.at[pl.ds(o, _SUB)]], ra.at[pl.ds(o, _SUB)],
                    sema))
                copies.append(pltpu.async_copy(
                    yb_hbm.at[ib.at[pl.ds(o, _SUB)]], rb.at[pl.ds(o, _SUB)],
                    semb))
            pltpu.sync_copy(z_hbm.at[0, pl.ds(tb, ctiles)],
                            rz.at[:, pl.ds(0, 8), :])
            pltpu.sync_copy(z_hbm.at[1, pl.ds(tb, ctiles)],
                            rz.at[:, pl.ds(8, 8), :])
            for c in copies:
                c.wait()

            @pl.loop(0, ctiles)
            def _tile(jt):
                ebase = jt * 128
                rzt = rz.at[jt]
                rtt = rt.at[jt]

                @pl.loop(0, 128, step=16)
                def _grp(jl0):
                    for dj in range(16):
                        jl = jl0 + dj
                        lane = jnp.full((16,), jl, jnp.int32)
                        e = ebase + jl
                        z16 = plsc.load_gather(rzt, [row_iota, lane])
                        v = jnp.maximum(ra[e] + rb[e] + z16, 0.0)
                        plsc.store_scatter(rtt, [row_iota, lane], v)

            pltpu.sync_copy(rt.at[:, pl.ds(0, 8), :],
                            out_hbm.at[0, pl.ds(tb, ctiles)])
            pltpu.sync_copy(rt.at[:, pl.ds(8, 8), :],
                            out_hbm.at[1, pl.ds(tb, ctiles)])

    return run(ya, yb, src, dst, zb)


def kernel(X, edge_index, edge_vals, W_pass, b_pass, W_self, b_self):
    n_nodes, d_n = X.shape
    n_edges = edge_vals.shape[0]
    d_out = W_pass.shape[0]

    src = edge_index[0].astype(jnp.int32)
    dst = edge_index[1].astype(jnp.int32)
    wa = W_pass[:, :d_n]
    wb = W_pass[:, d_n:]
    wya = (wa + wb) * 0.5
    wyb = (wb - wa) * 0.5
    bias = (b_pass + b_self).reshape(1, d_out)

    eye8 = jnp.eye(8, dtype=jnp.float32)
    wya_big = jnp.kron(eye8, wya.T)
    wyb_big = jnp.kron(eye8, wyb.T)
    bias_big = jnp.tile(bias, (1, 8))

    xr = X.reshape(n_nodes // 8, 8 * d_n)
    ya2, yb2 = pl.pallas_call(
        _y_body,
        out_shape=[jax.ShapeDtypeStruct((n_nodes // 8, 8 * d_out),
                                        jnp.float32)] * 2,
    )(xr, wya_big, wyb_big, bias_big)
    ya = ya2.reshape(n_nodes, d_out)
    yb = yb2.reshape(n_nodes, d_out)

    evt = edge_vals.T
    ntiles = n_edges // 128
    bt = _BC // 128
    zb = pl.pallas_call(
        _zt_body,
        grid=(n_edges // _BC,),
        in_specs=[
            pl.BlockSpec((d_out, _BC), lambda i: (0, i)),
            pl.BlockSpec(W_self.shape, lambda i: (0, 0)),
        ],
        out_specs=pl.BlockSpec((2, bt, 8, 128), lambda i: (0, i, 0, 0)),
        out_shape=jax.ShapeDtypeStruct((2, ntiles, 8, 128), jnp.float32),
    )(evt, W_self)

    out4 = _sc_combine(ya, yb, src, dst, zb)
    return out4.transpose(1, 3, 0, 2).reshape(n_edges, d_out)


# speedup vs baseline: 5.8203x; 1.2624x over previous
import functools

import jax
import jax.numpy as jnp
from jax import lax
from jax.experimental import pallas as pl
from jax.experimental.pallas import tpu as pltpu
from jax.experimental.pallas import tpu_sc as plsc

_NW = 32
_C = 512
_SUB = 128
_BC = 12800


def _y_body(x_ref, wya_ref, wyb_ref, b_ref, ya_ref, yb_ref):
    x = x_ref[...]
    dn = (((1,), (0,)), ((), ()))
    ya_ref[...] = lax.dot_general(x, wya_ref[...], dn,
                                  preferred_element_type=jnp.float32) + b_ref[...]
    yb_ref[...] = lax.dot_general(x, wyb_ref[...], dn,
                                  preferred_element_type=jnp.float32)


def _zt_body(evt_ref, w_ref, z_ref):
    zt = lax.dot_general(w_ref[...], evt_ref[...], (((1,), (0,)), ((), ())),
                         preferred_element_type=jnp.float32)
    bt = zt.shape[1] // 128
    z_ref[...] = zt.reshape(2, 8, bt, 128).transpose(0, 2, 1, 3)


def _sc_combine(ya, yb, src, dst, zb):
    n_edges = src.shape[0]
    d_out = ya.shape[1]
    nchunk = n_edges // _C
    ctiles = _C // 128
    ntiles = n_edges // 128
    mesh = plsc.VectorSubcoreMesh(core_axis_name="c", subcore_axis_name="s")

    @functools.partial(
        pl.kernel,
        mesh=mesh,
        compiler_params=pltpu.CompilerParams(use_tc_tiling_on_sc=False,
                                             needs_layout_passes=False),
        out_type=jax.ShapeDtypeStruct((2, ntiles, 8, 128), jnp.float32),
        scratch_types=[
            pltpu.VMEM((_C,), jnp.int32),
            pltpu.VMEM((_C,), jnp.int32),
            pltpu.VMEM((_C, d_out), jnp.float32),
            pltpu.VMEM((_C, d_out), jnp.float32),
            pltpu.VMEM((ctiles, 16, 128), jnp.float32),
            pltpu.VMEM((ctiles, 16, 128), jnp.float32),
            pltpu.VMEM((_C * 17,), jnp.float32),
            pltpu.SemaphoreType.DMA,
            pltpu.SemaphoreType.DMA,
        ],
    )
    def run(ya_hbm, yb_hbm, src_hbm, dst_hbm, z_hbm, out_hbm,
            ia, ib, ra, rb, rz, rt, rs, sema, semb):
        wid = lax.axis_index("s") * 2 + lax.axis_index("c")
        iota17 = lax.iota(jnp.int32, 16) * 17

        @pl.loop(wid, nchunk, step=_NW)
        def _chunk(t):
            base = pl.multiple_of(t * _C, 8)
            tb = pl.multiple_of(t * ctiles, 4)
            pltpu.sync_copy(src_hbm.at[pl.ds(base, _C)], ia)
            pltpu.sync_copy(dst_hbm.at[pl.ds(base, _C)], ib)
            copies = []
            for j in range(_C // _SUB):
                o = j * _SUB
                copies.append(pltpu.async_copy(
                    ya_hbm.at[ia.at[pl.ds(o, _SUB)]], ra.at[pl.ds(o, _SUB)],
                    sema))
                copies.append(pltpu.async_copy(
                    yb_hbm.at[ib.at[pl.ds(o, _SUB)]], rb.at[pl.ds(o, _SUB)],
                    semb))
            pltpu.sync_copy(z_hbm.at[0, pl.ds(tb, ctiles)],
                            rz.at[:, pl.ds(0, 8), :])
            pltpu.sync_copy(z_hbm.at[1, pl.ds(tb, ctiles)],
                            rz.at[:, pl.ds(8, 8), :])
            for c in copies:
                c.wait()

            @pl.loop(0, _C, step=8)
            def _s1(e0):
                for de in range(8):
                    e = e0 + de
                    rs[pl.ds(e * 17, 16)] = ra[e] + rb[e]

            @pl.loop(0, ctiles)
            def _tile(jt):
                for c in range(16):
                    for g in range(8):
                        e0 = jt * 128 + g * 16
                        idx = iota17 + (e0 * 17 + c)
                        w = plsc.load_gather(rs, [idx])
                        zrow = rz[jt, c, pl.ds(g * 16, 16)]
                        rt[jt, c, pl.ds(g * 16, 16)] = jnp.maximum(w + zrow,
                                                                   0.0)

            pltpu.sync_copy(rt.at[:, pl.ds(0, 8), :],
                            out_hbm.at[0, pl.ds(tb, ctiles)])
            pltpu.sync_copy(rt.at[:, pl.ds(8, 8), :],
                            out_hbm.at[1, pl.ds(tb, ctiles)])

    return run(ya, yb, src, dst, zb)


def kernel(X, edge_index, edge_vals, W_pass, b_pass, W_self, b_self):
    n_nodes, d_n = X.shape
    n_edges = edge_vals.shape[0]
    d_out = W_pass.shape[0]

    src = edge_index[0].astype(jnp.int32)
    dst = edge_index[1].astype(jnp.int32)
    wa = W_pass[:, :d_n]
    wb = W_pass[:, d_n:]
    wya = (wa + wb) * 0.5
    wyb = (wb - wa) * 0.5
    bias = (b_pass + b_self).reshape(1, d_out)

    eye8 = jnp.eye(8, dtype=jnp.float32)
    wya_big = jnp.kron(eye8, wya.T)
    wyb_big = jnp.kron(eye8, wyb.T)
    bias_big = jnp.tile(bias, (1, 8))

    xr = X.reshape(n_nodes // 8, 8 * d_n)
    ya2, yb2 = pl.pallas_call(
        _y_body,
        out_shape=[jax.ShapeDtypeStruct((n_nodes // 8, 8 * d_out),
                                        jnp.float32)] * 2,
    )(xr, wya_big, wyb_big, bias_big)
    ya = ya2.reshape(n_nodes, d_out)
    yb = yb2.reshape(n_nodes, d_out)

    evt = edge_vals.T
    ntiles = n_edges // 128
    bt = _BC // 128
    zb = pl.pallas_call(
        _zt_body,
        grid=(n_edges // _BC,),
        in_specs=[
            pl.BlockSpec((d_out, _BC), lambda i: (0, i)),
            pl.BlockSpec(W_self.shape, lambda i: (0, 0)),
        ],
        out_specs=pl.BlockSpec((2, bt, 8, 128), lambda i: (0, i, 0, 0)),
        out_shape=jax.ShapeDtypeStruct((2, ntiles, 8, 128), jnp.float32),
    )(evt, W_self)

    out4 = _sc_combine(ya, yb, src, dst, zb)
    return out4.transpose(1, 3, 0, 2).reshape(n_edges, d_out)


# speedup vs baseline: 10.0106x; 1.7199x over previous
import functools

import jax
import jax.numpy as jnp
from jax import lax
from jax.experimental import pallas as pl
from jax.experimental.pallas import tpu as pltpu
from jax.experimental.pallas import tpu_sc as plsc

_NW = 32
_C = 512
_SUB = 128
_BC = 12800


def _y_body(x_ref, wya_ref, wyb_ref, b_ref, ya_ref, yb_ref):
    x = x_ref[...]
    dn = (((1,), (0,)), ((), ()))
    ya_ref[...] = lax.dot_general(x, wya_ref[...], dn,
                                  preferred_element_type=jnp.float32) + b_ref[...]
    yb_ref[...] = lax.dot_general(x, wyb_ref[...], dn,
                                  preferred_element_type=jnp.float32)


def _zt_body(evt_ref, w_ref, z_ref):
    zt = lax.dot_general(w_ref[...], evt_ref[...], (((1,), (0,)), ((), ())),
                         preferred_element_type=jnp.float32)
    bt = zt.shape[1] // 128
    z_ref[...] = zt.reshape(2, 8, bt, 128).transpose(0, 2, 1, 3)


def _sc_combine(ya, yb, src, dst, zb):
    n_edges = src.shape[0]
    d_out = ya.shape[1]
    nchunk = n_edges // _C
    ctiles = _C // 128
    ntiles = n_edges // 128
    mesh = plsc.VectorSubcoreMesh(core_axis_name="c", subcore_axis_name="s")

    @functools.partial(
        pl.kernel,
        mesh=mesh,
        compiler_params=pltpu.CompilerParams(use_tc_tiling_on_sc=False,
                                             needs_layout_passes=False),
        out_type=jax.ShapeDtypeStruct((2, ntiles, 8, 128), jnp.float32),
        scratch_types=[
            pltpu.VMEM((_C,), jnp.int32),
            pltpu.VMEM((_C,), jnp.int32),
            pltpu.VMEM((_C, d_out), jnp.float32),
            pltpu.VMEM((_C, d_out), jnp.float32),
            pltpu.VMEM((ctiles, 16, 128), jnp.float32),
            pltpu.VMEM((ctiles, 16, 128), jnp.float32),
            pltpu.VMEM((_C * 17,), jnp.float32),
            pltpu.SemaphoreType.DMA,
            pltpu.SemaphoreType.DMA,
        ],
    )
    def run(ya_hbm, yb_hbm, src_hbm, dst_hbm, z_hbm, out_hbm,
            ia, ib, ra, rb, rz, rt, rs, sema, semb):
        wid = lax.axis_index("s") * 2 + lax.axis_index("c")
        iota17 = lax.iota(jnp.int32, 16) * 17

        @pl.loop(wid, nchunk, step=_NW)
        def _chunk(t):
            base = pl.multiple_of(t * _C, 8)
            tb = pl.multiple_of(t * ctiles, 4)
            pltpu.sync_copy(src_hbm.at[pl.ds(base, _C)], ia)
            pltpu.sync_copy(dst_hbm.at[pl.ds(base, _C)], ib)
            copies = []
            for j in range(_C // _SUB):
                o = j * _SUB
                copies.append(pltpu.async_copy(
                    ya_hbm.at[ia.at[pl.ds(o, _SUB)]], ra.at[pl.ds(o, _SUB)],
                    sema))
                copies.append(pltpu.async_copy(
                    yb_hbm.at[ib.at[pl.ds(o, _SUB)]], rb.at[pl.ds(o, _SUB)],
                    semb))
            pltpu.sync_copy(z_hbm.at[0, pl.ds(tb, ctiles)],
                            rz.at[:, pl.ds(0, 8), :])
            pltpu.sync_copy(z_hbm.at[1, pl.ds(tb, ctiles)],
                            rz.at[:, pl.ds(8, 8), :])
            for c in copies:
                c.wait()

            @plsc.parallel_loop(0, _C, unroll=8)
            def _s1(e):
                rs[pl.ds(e * 17, 16)] = ra[e] + rb[e]

            @plsc.parallel_loop(0, ctiles * 128, unroll=8)
            def _tile(k):
                jt = k // 128
                m = k % 128
                c = m // 8
                g = m % 8
                e0 = jt * 128 + g * 16
                idx = iota17 + (e0 * 17 + c)
                w = plsc.load_gather(rs, [idx])
                zrow = rz[jt, c, pl.ds(g * 16, 16)]
                rt[jt, c, pl.ds(g * 16, 16)] = jnp.maximum(w + zrow, 0.0)

            pltpu.sync_copy(rt.at[:, pl.ds(0, 8), :],
                            out_hbm.at[0, pl.ds(tb, ctiles)])
            pltpu.sync_copy(rt.at[:, pl.ds(8, 8), :],
                            out_hbm.at[1, pl.ds(tb, ctiles)])

    return run(ya, yb, src, dst, zb)


def kernel(X, edge_index, edge_vals, W_pass, b_pass, W_self, b_self):
    n_nodes, d_n = X.shape
    n_edges = edge_vals.shape[0]
    d_out = W_pass.shape[0]

    src = edge_index[0].astype(jnp.int32)
    dst = edge_index[1].astype(jnp.int32)
    wa = W_pass[:, :d_n]
    wb = W_pass[:, d_n:]
    wya = (wa + wb) * 0.5
    wyb = (wb - wa) * 0.5
    bias = (b_pass + b_self).reshape(1, d_out)

    eye8 = jnp.eye(8, dtype=jnp.float32)
    wya_big = jnp.kron(eye8, wya.T)
    wyb_big = jnp.kron(eye8, wyb.T)
    bias_big = jnp.tile(bias, (1, 8))

    xr = X.reshape(n_nodes // 8, 8 * d_n)
    ya2, yb2 = pl.pallas_call(
        _y_body,
        out_shape=[jax.ShapeDtypeStruct((n_nodes // 8, 8 * d_out),
                                        jnp.float32)] * 2,
    )(xr, wya_big, wyb_big, bias_big)
    ya = ya2.reshape(n_nodes, d_out)
    yb = yb2.reshape(n_nodes, d_out)

    evt = edge_vals.T
    ntiles = n_edges // 128
    bt = _BC // 128
    zb = pl.pallas_call(
        _zt_body,
        grid=(n_edges // _BC,),
        in_specs=[
            pl.BlockSpec((d_out, _BC), lambda i: (0, i)),
            pl.BlockSpec(W_self.shape, lambda i: (0, 0)),
        ],
        out_specs=pl.BlockSpec((2, bt, 8, 128), lambda i: (0, i, 0, 0)),
        out_shape=jax.ShapeDtypeStruct((2, ntiles, 8, 128), jnp.float32),
    )(evt, W_self)

    out4 = _sc_combine(ya, yb, src, dst, zb)
    return out4.transpose(1, 3, 0, 2).reshape(n_edges, d_out)


# speedup vs baseline: 12.8281x; 1.2815x over previous
import functools

import jax
import jax.numpy as jnp
from jax import lax
from jax.experimental import pallas as pl
from jax.experimental.pallas import tpu as pltpu
from jax.experimental.pallas import tpu_sc as plsc

_NW = 32
_C = 512
_SUB = 128
_BC = 12800


def _y_body(x_ref, wya_ref, wyb_ref, b_ref, ya_ref, yb_ref):
    x = x_ref[...]
    dn = (((1,), (0,)), ((), ()))
    ya_ref[...] = lax.dot_general(x, wya_ref[...], dn,
                                  preferred_element_type=jnp.float32) + b_ref[...]
    yb_ref[...] = lax.dot_general(x, wyb_ref[...], dn,
                                  preferred_element_type=jnp.float32)


def _zt_body(evt_ref, w_ref, z_ref):
    zt = lax.dot_general(w_ref[...], evt_ref[...], (((1,), (0,)), ((), ())),
                         preferred_element_type=jnp.float32)
    bt = zt.shape[1] // 128
    z_ref[...] = zt.reshape(2, 8, bt, 128).transpose(0, 2, 1, 3)


def _sc_combine(ya, yb, src, dst, zb):
    n_edges = src.shape[0]
    d_out = ya.shape[1]
    nchunk = n_edges // _C
    ctiles = _C // 128
    ntiles = n_edges // 128
    mesh = plsc.VectorSubcoreMesh(core_axis_name="c", subcore_axis_name="s")

    @functools.partial(
        pl.kernel,
        mesh=mesh,
        compiler_params=pltpu.CompilerParams(use_tc_tiling_on_sc=False,
                                             needs_layout_passes=False),
        out_type=jax.ShapeDtypeStruct((2, ntiles, 8, 128), jnp.float32),
        scratch_types=[
            pltpu.VMEM((2, _C), jnp.int32),
            pltpu.VMEM((2, _C), jnp.int32),
            pltpu.VMEM((2, _C, d_out), jnp.float32),
            pltpu.VMEM((2, _C, d_out), jnp.float32),
            pltpu.VMEM((2, ctiles, 16, 128), jnp.float32),
            pltpu.VMEM((2, ctiles, 16, 128), jnp.float32),
            pltpu.VMEM((_C * 17,), jnp.float32),
            pltpu.SemaphoreType.DMA,
            pltpu.SemaphoreType.DMA,
            pltpu.SemaphoreType.DMA,
            pltpu.SemaphoreType.DMA,
            pltpu.SemaphoreType.DMA,
            pltpu.SemaphoreType.DMA,
        ],
    )
    def run(ya_hbm, yb_hbm, src_hbm, dst_hbm, z_hbm, out_hbm,
            ia2, ib2, ra2, rb2, rz2, rt2, rs,
            sa0, sb0, sz0, sa1, sb1, sz1):
        wid = lax.axis_index("s") * 2 + lax.axis_index("c")
        iota17 = lax.iota(jnp.int32, 16) * 17
        sems = [(sa0, sb0, sz0), (sa1, sb1, sz1)]
        npair = (nchunk // _NW + 2) // 2
        nk = (nchunk - wid + _NW - 1) // _NW

        def start_load(s, t):
            ia, ib, ra, rb, rz = ia2.at[s], ib2.at[s], ra2.at[s], rb2.at[s], rz2.at[s]
            sema, semb, semz = sems[s]
            base = pl.multiple_of(t * _C, 8)
            tb = pl.multiple_of(t * ctiles, 4)
            pltpu.sync_copy(src_hbm.at[pl.ds(base, _C)], ia)
            pltpu.sync_copy(dst_hbm.at[pl.ds(base, _C)], ib)
            for j in range(_C // _SUB):
                o = j * _SUB
                pltpu.async_copy(ya_hbm.at[ia.at[pl.ds(o, _SUB)]],
                                 ra.at[pl.ds(o, _SUB)], sema)
                pltpu.async_copy(yb_hbm.at[ib.at[pl.ds(o, _SUB)]],
                                 rb.at[pl.ds(o, _SUB)], semb)
            pltpu.async_copy(z_hbm.at[0, pl.ds(tb, ctiles)],
                             rz.at[:, pl.ds(0, 8), :], semz)
            pltpu.async_copy(z_hbm.at[1, pl.ds(tb, ctiles)],
                             rz.at[:, pl.ds(8, 8), :], semz)

        def compute_store(s, t):
            ia, ib, ra, rb = ia2.at[s], ib2.at[s], ra2.at[s], rb2.at[s]
            rz, rt = rz2.at[s], rt2.at[s]
            sema, semb, semz = sems[s]
            tb = pl.multiple_of(t * ctiles, 4)
            for j in range(_C // _SUB):
                o = j * _SUB
                pltpu.make_async_copy(ya_hbm.at[ia.at[pl.ds(o, _SUB)]],
                                      ra.at[pl.ds(o, _SUB)], sema).wait()
                pltpu.make_async_copy(yb_hbm.at[ib.at[pl.ds(o, _SUB)]],
                                      rb.at[pl.ds(o, _SUB)], semb).wait()
            pltpu.make_async_copy(z_hbm.at[0, pl.ds(tb, ctiles)],
                                  rz.at[:, pl.ds(0, 8), :], semz).wait()
            pltpu.make_async_copy(z_hbm.at[1, pl.ds(tb, ctiles)],
                                  rz.at[:, pl.ds(8, 8), :], semz).wait()

            @plsc.parallel_loop(0, _C, unroll=8)
            def _s1(e):
                rs[pl.ds(e * 17, 16)] = ra[e] + rb[e]

            @plsc.parallel_loop(0, ctiles * 128, unroll=8)
            def _tile(k):
                jt = k // 128
                m = k % 128
                c = m // 8
                g = m % 8
                e0 = jt * 128 + g * 16
                idx = iota17 + (e0 * 17 + c)
                w = plsc.load_gather(rs, [idx])
                zrow = rz[jt, c, pl.ds(g * 16, 16)]
                rt[jt, c, pl.ds(g * 16, 16)] = jnp.maximum(w + zrow, 0.0)

            pltpu.sync_copy(rt.at[:, pl.ds(0, 8), :],
                            out_hbm.at[0, pl.ds(tb, ctiles)])
            pltpu.sync_copy(rt.at[:, pl.ds(8, 8), :],
                            out_hbm.at[1, pl.ds(tb, ctiles)])

        start_load(0, wid)

        @pl.loop(0, npair)
        def _pair(kp):
            k_b = 2 * kp + 1
            t_a = wid + 2 * kp * _NW
            t_b = wid + k_b * _NW

            @pl.when(k_b < nk)
            def _():
                start_load(1, t_b)

            compute_store(0, t_a)

            @pl.when(k_b + 1 < nk)
            def _():
                start_load(0, t_b + _NW)

            @pl.when(k_b < nk)
            def _():
                compute_store(1, t_b)

    return run(ya, yb, src, dst, zb)


def kernel(X, edge_index, edge_vals, W_pass, b_pass, W_self, b_self):
    n_nodes, d_n = X.shape
    n_edges = edge_vals.shape[0]
    d_out = W_pass.shape[0]

    src = edge_index[0].astype(jnp.int32)
    dst = edge_index[1].astype(jnp.int32)
    wa = W_pass[:, :d_n]
    wb = W_pass[:, d_n:]
    wya = (wa + wb) * 0.5
    wyb = (wb - wa) * 0.5
    bias = (b_pass + b_self).reshape(1, d_out)

    eye8 = jnp.eye(8, dtype=jnp.float32)
    wya_big = jnp.kron(eye8, wya.T)
    wyb_big = jnp.kron(eye8, wyb.T)
    bias_big = jnp.tile(bias, (1, 8))

    xr = X.reshape(n_nodes // 8, 8 * d_n)
    ya2, yb2 = pl.pallas_call(
        _y_body,
        out_shape=[jax.ShapeDtypeStruct((n_nodes // 8, 8 * d_out),
                                        jnp.float32)] * 2,
    )(xr, wya_big, wyb_big, bias_big)
    ya = ya2.reshape(n_nodes, d_out)
    yb = yb2.reshape(n_nodes, d_out)

    evt = edge_vals.T
    ntiles = n_edges // 128
    bt = _BC // 128
    zb = pl.pallas_call(
        _zt_body,
        grid=(n_edges // _BC,),
        in_specs=[
            pl.BlockSpec((d_out, _BC), lambda i: (0, i)),
            pl.BlockSpec(W_self.shape, lambda i: (0, 0)),
        ],
        out_specs=pl.BlockSpec((2, bt, 8, 128), lambda i: (0, i, 0, 0)),
        out_shape=jax.ShapeDtypeStruct((2, ntiles, 8, 128), jnp.float32),
    )(evt, W_self)

    out4 = _sc_combine(ya, yb, src, dst, zb)
    return out4.transpose(1, 3, 0, 2).reshape(n_edges, d_out)
